# speedup vs baseline: 2.6955x; 2.6955x over previous
import functools

import jax
import jax.numpy as jnp
from jax import lax
from jax.experimental import pallas as pl
from jax.experimental.pallas import tpu as pltpu
from jax.experimental.pallas import tpu_sc as plsc

N_CODES = 1024
N_ROWS = 131072
D = 256

_info = plsc.get_sparse_core_info()
NC, NS = _info.num_cores, _info.num_subcores
NW = NC * NS
BPW = N_ROWS // NW
C = 128
NCHUNK = BPW // C
NBUF = 2


def _body(table_hbm, codes_hbm, out_hbm, idx_v, rows_v, gsem, ssem):
    wid = lax.axis_index("s") * NC + lax.axis_index("c")
    base = wid * BPW
    pltpu.sync_copy(codes_hbm.at[pl.ds(base, BPW)], idx_v)

    def gather_start(chunk, slot):
        pltpu.async_copy(
            table_hbm.at[idx_v.at[pl.ds(chunk * C, C)]], rows_v.at[slot], gsem)

    def gather_wait(slot):
        pltpu.make_async_copy(
            table_hbm.at[idx_v.at[pl.ds(0, C)]], rows_v.at[slot], gsem).wait()

    def scatter_start(chunk, slot):
        pltpu.async_copy(
            rows_v.at[slot], out_hbm.at[pl.ds(base + chunk * C, C)], ssem)

    def scatter_wait(slot):
        pltpu.make_async_copy(
            rows_v.at[slot], out_hbm.at[pl.ds(base, C)], ssem).wait()

    gather_start(0, 0)

    def step(i, carry):
        slot = lax.rem(i, 2)
        nslot = lax.rem(i + 1, 2)

        @pl.when(i >= 1)
        def _():
            scatter_wait(nslot)

        @pl.when(i + 1 < NCHUNK)
        def _():
            gather_start(i + 1, nslot)

        gather_wait(slot)
        scatter_start(i, slot)
        return carry

    lax.fori_loop(0, NCHUNK, step, 0)
    scatter_wait(lax.rem(jnp.int32(NCHUNK - 1), 2))


def _dequant(q2d, codes):
    run = functools.partial(
        pl.kernel,
        mesh=plsc.VectorSubcoreMesh(core_axis_name="c", subcore_axis_name="s"),
        out_type=jax.ShapeDtypeStruct((N_ROWS, D), jnp.float32),
        scratch_types=[
            pltpu.VMEM((BPW,), jnp.int32),
            pltpu.VMEM((NBUF, C, D), jnp.float32),
            pltpu.SemaphoreType.DMA,
            pltpu.SemaphoreType.DMA,
        ],
    )(_body)
    return run(q2d, codes)


def kernel(quantized, codes):
    trailing = quantized.shape[1:]
    q2d = quantized.reshape(quantized.shape[0], -1)
    full = _dequant(q2d, codes)
    return full.reshape((-1, *trailing))
